# Initial kernel scaffold; baseline (speedup 1.0000x reference)
import jax
import jax.numpy as jnp
from jax.experimental import pallas as pl


def kernel(x, rule_feat_idxs, rule_threshs, rule_signs, rule_masks, premise_params, consequent_params, attention_weights, interaction_pairs):
    raise NotImplementedError("write your pallas kernel here")


# speedup vs baseline: 4.5395x; 4.5395x over previous
import jax
import jax.numpy as jnp
from jax.experimental import pallas as pl


def _pad_rows(arr, rows=8):
    return jnp.pad(arr, ((0, rows - arr.shape[0]), (0, 0)))


def _anfis_body(x_ref, idx_ref, fpar_ref, wc_ref, d_ref, pairs_ref, aw_ref,
                o_ref, *, F, R, L, P):
    xa = x_ref[...] * aw_ref[0:1, :]

    idx = idx_ref[0:1, :]
    a = fpar_ref[0:1, :]
    b = fpar_ref[1:2, :]
    m = fpar_ref[2:3, :]
    iota = jax.lax.broadcasted_iota(jnp.int32, (F, L * R), 0)
    w_sel = jnp.where(iota == idx, a, 0.0)
    z = jnp.dot(xa, w_sel, preferred_element_type=jnp.float32) + b
    mf = jax.nn.sigmoid(z)
    mfm = mf * m + (1.0 - m)
    firing = mfm[:, 0:R]
    for l in range(1, L):
        firing = firing * mfm[:, l * R:(l + 1) * R]

    i1 = pairs_ref[0:1, :]
    i2 = pairs_ref[1:2, :]
    iota_p = jax.lax.broadcasted_iota(jnp.int32, (F, P), 0)
    oh1 = (iota_p == i1).astype(jnp.float32)
    oh2 = (iota_p == i2).astype(jnp.float32)
    g1 = jnp.dot(xa, oh1, preferred_element_type=jnp.float32)
    g2 = jnp.dot(xa, oh2, preferred_element_type=jnp.float32)
    feats = jnp.concatenate([xa, xa * xa, g1 * g2], axis=1)
    ro = jnp.dot(feats, wc_ref[...], preferred_element_type=jnp.float32)
    ro = ro + d_ref[0:1, :]

    num = jnp.sum(firing * ro, axis=1, keepdims=True)
    den = jnp.sum(firing, axis=1, keepdims=True) + 1e-8
    o_ref[...] = num / den


def kernel(x, rule_feat_idxs, rule_threshs, rule_signs, rule_masks,
           premise_params, consequent_params, attention_weights,
           interaction_pairs):
    B, F = x.shape
    R, L = rule_feat_idxs.shape
    P = interaction_pairs.shape[0]
    LR = L * R

    idx_flat = rule_feat_idxs.T.reshape(1, LR).astype(jnp.int32)
    beta = premise_params[None, :]
    a = (rule_signs.T * beta).reshape(1, LR)
    b = (-rule_signs.T * rule_threshs.T * beta).reshape(1, LR)
    m = rule_masks.T.reshape(1, LR)
    fpar = _pad_rows(jnp.concatenate([a, b, m], axis=0))
    idx_p = _pad_rows(idx_flat)

    wc = consequent_params[:, :2 * F + P].T
    d_p = _pad_rows(consequent_params[:, 2 * F + P:].T)
    pairs_p = _pad_rows(interaction_pairs.T.astype(jnp.int32))
    aw_p = _pad_rows(attention_weights[None, :])

    bB = 256
    grid = (B // bB,)
    const = lambda shape: pl.BlockSpec(shape, lambda i: (0, 0))

    import functools
    body = functools.partial(_anfis_body, F=F, R=R, L=L, P=P)
    y = pl.pallas_call(
        body,
        grid=grid,
        in_specs=[
            pl.BlockSpec((bB, F), lambda i: (i, 0)),
            const((8, LR)),
            const((8, LR)),
            const((2 * F + P, R)),
            const((8, R)),
            const((8, P)),
            const((8, F)),
        ],
        out_specs=pl.BlockSpec((bB, 1), lambda i: (i, 0)),
        out_shape=jax.ShapeDtypeStruct((B, 1), jnp.float32),
    )(x, idx_p, fpar, wc, d_p, pairs_p, aw_p)
    return y


# speedup vs baseline: 4.9808x; 1.0972x over previous
import jax
import jax.numpy as jnp
from jax.experimental import pallas as pl
from jax.experimental.pallas import tpu as pltpu


def _pad_rows(arr, rows=8):
    return jnp.pad(arr, ((0, rows - arr.shape[0]), (0, 0)))


def _anfis_body(x_ref, idx_ref, fpar_ref, wc_ref, d_ref, pairs_ref, aw_ref,
                o_ref, wsel_ref, ohp_ref, *, F, R, L, P):
    @pl.when(pl.program_id(0) == 0)
    def _build_onehots():
        idx = idx_ref[0:1, :]
        a = fpar_ref[0:1, :]
        iota = jax.lax.broadcasted_iota(jnp.int32, (F, L * R), 0)
        wsel_ref[...] = jnp.where(iota == idx, a, 0.0)
        i1 = pairs_ref[0:1, :]
        i2 = pairs_ref[1:2, :]
        iota_p = jax.lax.broadcasted_iota(jnp.int32, (F, P), 0)
        ohp_ref[:, 0:P] = (iota_p == i1).astype(jnp.float32)
        ohp_ref[:, P:2 * P] = (iota_p == i2).astype(jnp.float32)

    xa = x_ref[...] * aw_ref[0:1, :]
    b = fpar_ref[1:2, :]
    m = fpar_ref[2:3, :]
    z = jnp.dot(xa, wsel_ref[...], preferred_element_type=jnp.float32) + b
    mf = jax.nn.sigmoid(z)
    mfm = mf * m + (1.0 - m)
    firing = mfm[:, 0:R]
    for l in range(1, L):
        firing = firing * mfm[:, l * R:(l + 1) * R]

    g12 = jnp.dot(xa, ohp_ref[...], preferred_element_type=jnp.float32)
    inter = g12[:, 0:P] * g12[:, P:2 * P]
    feats = jnp.concatenate([xa, xa * xa, inter], axis=1)
    ro = jnp.dot(feats, wc_ref[...], preferred_element_type=jnp.float32)
    ro = ro + d_ref[0:1, :]

    num = jnp.sum(firing * ro, axis=1, keepdims=True)
    den = jnp.sum(firing, axis=1, keepdims=True) + 1e-8
    o_ref[...] = num / den


def kernel(x, rule_feat_idxs, rule_threshs, rule_signs, rule_masks,
           premise_params, consequent_params, attention_weights,
           interaction_pairs):
    B, F = x.shape
    R, L = rule_feat_idxs.shape
    P = interaction_pairs.shape[0]
    LR = L * R

    idx_flat = rule_feat_idxs.T.reshape(1, LR).astype(jnp.int32)
    beta = premise_params[None, :]
    a = (rule_signs.T * beta).reshape(1, LR)
    b = (-rule_signs.T * rule_threshs.T * beta).reshape(1, LR)
    m = rule_masks.T.reshape(1, LR)
    fpar = _pad_rows(jnp.concatenate([a, b, m], axis=0))
    idx_p = _pad_rows(idx_flat)

    wc = consequent_params[:, :2 * F + P].T
    d_p = _pad_rows(consequent_params[:, 2 * F + P:].T)
    pairs_p = _pad_rows(interaction_pairs.T.astype(jnp.int32))
    aw_p = _pad_rows(attention_weights[None, :])

    bB = 512
    grid = (B // bB,)
    const = lambda shape: pl.BlockSpec(shape, lambda i: (0, 0))

    import functools
    body = functools.partial(_anfis_body, F=F, R=R, L=L, P=P)
    y = pl.pallas_call(
        body,
        grid=grid,
        in_specs=[
            pl.BlockSpec((bB, F), lambda i: (i, 0)),
            const((8, LR)),
            const((8, LR)),
            const((2 * F + P, R)),
            const((8, R)),
            const((8, P)),
            const((8, F)),
        ],
        out_specs=pl.BlockSpec((bB, 1), lambda i: (i, 0)),
        out_shape=jax.ShapeDtypeStruct((B, 1), jnp.float32),
        scratch_shapes=[
            pltpu.VMEM((F, LR), jnp.float32),
            pltpu.VMEM((F, 2 * P), jnp.float32),
        ],
    )(x, idx_p, fpar, wc, d_p, pairs_p, aw_p)
    return y
